# Initial kernel scaffold; baseline (speedup 1.0000x reference)
import jax
import jax.numpy as jnp
from jax.experimental import pallas as pl


def kernel(x, params):
    raise NotImplementedError("write your pallas kernel here")


# speedup vs baseline: 3.1541x; 3.1541x over previous
import functools

import jax
import jax.numpy as jnp
from jax import lax
from jax.experimental import pallas as pl
from jax.experimental.pallas import tpu as pltpu
from jax.experimental.pallas import tpu_sc as plsc

_POOL = 4
_K = 16
_NW = 32


def _rup(n, m):
    return ((n + m - 1) // m) * m


def _knn_body(qc_ref, rc_ref, out_ref, *, k, nr):
    bq = out_ref.shape[0]
    d = None
    for dim in range(3):
        qv = qc_ref[dim, :]
        rv = rc_ref[dim, :]
        diff = qv[:, None] - rv[None, :]
        sq = diff * diff
        d = sq if d is None else d + sq
    iota = lax.broadcasted_iota(jnp.int32, (bq, nr), 1)
    cols = []
    for j in range(k):
        m = jnp.min(d, axis=1, keepdims=True)
        ii = jnp.min(jnp.where(d == m, iota, jnp.int32(nr)), axis=1)
        cols.append(ii[:, None])
        if j + 1 < k:
            d = jnp.where(iota == ii[:, None], jnp.float32(jnp.inf), d)
    out_ref[...] = jnp.concatenate(cols, axis=1) if k > 1 else cols[0]


def _knn(qt, rt, k, bq=128):
    nq, nr = qt.shape[1], rt.shape[1]
    return pl.pallas_call(
        functools.partial(_knn_body, k=k, nr=nr),
        grid=(nq // bq,),
        in_specs=[pl.BlockSpec((4, bq), lambda i: (0, i)),
                  pl.BlockSpec((4, nr), lambda i: (0, 0))],
        out_specs=pl.BlockSpec((bq, k), lambda i: (i, 0)),
        out_shape=jax.ShapeDtypeStruct((nq, k), jnp.int32),
    )(qt, rt)


def _pad_coords_t(c, n_pad, fill):
    n = c.shape[0]
    ct = jnp.concatenate([c.T, jnp.zeros((1, n), jnp.float32)], axis=0)
    return jnp.pad(ct, ((0, 0), (0, n_pad - n)), constant_values=fill)


def _mm_body(x_ref, w_ref, b_ref, o_ref, *, relu):
    acc = jnp.dot(x_ref[...], w_ref[...],
                  preferred_element_type=jnp.float32) + b_ref[...]
    o_ref[...] = jnp.maximum(acc, 0.0) if relu else acc


def _mm(x, w, b, relu=False, bm=256):
    m, kd = x.shape
    n = w.shape[1]
    return pl.pallas_call(
        functools.partial(_mm_body, relu=relu),
        grid=(m // bm,),
        in_specs=[pl.BlockSpec((bm, kd), lambda i: (i, 0)),
                  pl.BlockSpec((kd, n), lambda i: (0, 0)),
                  pl.BlockSpec((1, n), lambda i: (0, 0))],
        out_specs=pl.BlockSpec((bm, n), lambda i: (i, 0)),
        out_shape=jax.ShapeDtypeStruct((m, n), jnp.float32),
    )(x, w, b.reshape(1, -1))


def _glayer_body(x_ref, c_ref, wf_ref, wr_ref, b_ref, g_ref, qp_ref):
    t = jnp.dot(c_ref[...], wr_ref[...], preferred_element_type=jnp.float32)
    g = jnp.dot(x_ref[...], wf_ref[...], preferred_element_type=jnp.float32)
    g_ref[...] = g + t + b_ref[...]
    qp_ref[...] = t


def _glayer(x, c4, w, b, bm=256):
    m, kd = x.shape
    n = w.shape[1]
    wf = w[:kd]
    wr = jnp.pad(w[kd:kd + 3], ((0, 1), (0, 0)))
    return pl.pallas_call(
        _glayer_body,
        grid=(m // bm,),
        in_specs=[pl.BlockSpec((bm, kd), lambda i: (i, 0)),
                  pl.BlockSpec((bm, 4), lambda i: (i, 0)),
                  pl.BlockSpec((kd, n), lambda i: (0, 0)),
                  pl.BlockSpec((4, n), lambda i: (0, 0)),
                  pl.BlockSpec((1, n), lambda i: (0, 0))],
        out_specs=[pl.BlockSpec((bm, n), lambda i: (i, 0)),
                   pl.BlockSpec((bm, n), lambda i: (i, 0))],
        out_shape=[jax.ShapeDtypeStruct((m, n), jnp.float32),
                   jax.ShapeDtypeStruct((m, n), jnp.float32)],
    )(x, c4, wf, wr, b.reshape(1, -1))


def _pad_coords4(c, n_pad):
    n = c.shape[0]
    return jnp.pad(c, ((0, n_pad - n), (0, 1)))


def _sc_gather_max(g, idx_flat, qp, k, gq):
    nq_pad, c = qp.shape
    nqw = nq_pad // _NW
    ngroups = nqw // gq
    nchunk = c // 16
    mesh = plsc.VectorSubcoreMesh(core_axis_name="c", subcore_axis_name="s")

    @functools.partial(
        pl.kernel, mesh=mesh,
        out_type=jax.ShapeDtypeStruct((nq_pad, c), jnp.float32),
        scratch_types=[
            pltpu.VMEM((gq * k,), jnp.int32),
            pltpu.VMEM((gq * k, c), jnp.float32),
            pltpu.VMEM((gq, c), jnp.float32),
            pltpu.VMEM((gq, c), jnp.float32),
            pltpu.SemaphoreType.DMA,
        ],
    )
    def run(g_hbm, idx_hbm, qp_hbm, out_hbm, idx_v, rows_v, qp_v, out_v, sem):
        wid = lax.axis_index("s") * 2 + lax.axis_index("c")
        base_q = wid * nqw

        def group(gi, _):
            q0 = base_q + gi * gq
            pltpu.sync_copy(idx_hbm.at[pl.ds(q0 * k, gq * k)], idx_v)
            pltpu.async_copy(g_hbm.at[idx_v], rows_v, sem).wait()
            pltpu.sync_copy(qp_hbm.at[pl.ds(q0, gq)], qp_v)

            def qloop(qq, _):
                def cloop(cc, _):
                    sl = pl.ds(cc * 16, 16)
                    acc = rows_v[qq * k, sl]
                    for kk in range(1, k):
                        acc = jnp.maximum(acc, rows_v[qq * k + kk, sl])
                    out_v[qq, sl] = jnp.maximum(acc - qp_v[qq, sl], 0.0)
                    return 0
                return lax.fori_loop(0, nchunk, cloop, 0)

            lax.fori_loop(0, gq, qloop, 0)
            pltpu.sync_copy(out_v, out_hbm.at[pl.ds(q0, gq)])
            return 0

        lax.fori_loop(0, ngroups, group, 0)

    return run(g, idx_flat, qp)


def kernel(x, params):
    n0 = x.shape[0]
    coords = [x[:, :3]]
    for _ in range(2):
        coords.append(coords[-1][::_POOL])
    n = [c.shape[0] for c in coords]
    npad = [_rup(v, 512) for v in n]
    feats = x[:, 3:]

    qts = [_pad_coords_t(c, p, 0.0) for c, p in zip(coords, npad)]
    rts = [_pad_coords_t(c, _rup(v, 128), 1e18) for c, v in zip(coords, n)]
    neigh = [_knn(qts[i], rts[i], _K) for i in range(3)]
    up = [_knn(qts[i], rts[i + 1], 1) for i in range(2)]
    sub = [neigh[i][::_POOL] for i in range(2)]

    c4 = [_pad_coords4(c, p) for c, p in zip(coords, npad)]

    def pad_rows(a, rows):
        return jnp.pad(a, ((0, rows - a.shape[0]), (0, 0)))

    enc, enc_b = params["enc"], params["enc_b"]
    pool, pool_b = params["pool"], params["pool_b"]
    dec, dec_b = params["dec"], params["dec_b"]

    level_feats = []
    X = pad_rows(feats, npad[0])
    for i in range(3):
        for l in range(i + 1):
            G, Qp = _glayer(X, c4[i], enc[i][l], enc_b[i][l])
            X = _sc_gather_max(G, neigh[i].reshape(-1), Qp, _K, 8)
        level_feats.append(X)
        if i < 2:
            G, Qp = _glayer(X, c4[i], pool[i][0], pool_b[i][0])
            sub_idx = pad_rows(sub[i], npad[i + 1]).reshape(-1)
            Qp_sub = pad_rows(Qp[::_POOL], npad[i + 1])
            X = _sc_gather_max(G, sub_idx, Qp_sub, _K, 8)
            for l in range(1, i + 1):
                X = _mm(X, pool[i][l], pool_b[i][l], relu=True)

    F = level_feats[2]
    for i in (1, 0):
        cu = dec[i].shape[0] - level_feats[i].shape[1]
        A = _mm(F, dec[i][:cu], jnp.zeros((dec[i].shape[1],), jnp.float32))
        Bneg = _mm(level_feats[i], -dec[i][cu:], -dec_b[i])
        F = _sc_gather_max(A, up[i].reshape(-1), Bneg, 1, 16)

    return F[:n0]
